# Initial kernel scaffold; baseline (speedup 1.0000x reference)
import jax
import jax.numpy as jnp
from jax.experimental import pallas as pl


def kernel(user, item, user_factors, item_factors, item_implicit_factors):
    raise NotImplementedError("write your pallas kernel here")


# speedup vs baseline: 1.6750x; 1.6750x over previous
import functools

import jax
import jax.numpy as jnp
from jax import lax
from jax.experimental import pallas as pl
from jax.experimental.pallas import tpu as pltpu
from jax.experimental.pallas import tpu_sc as plsc

F = 64
NW = 32
CHUNK = 128
LANES = 16


def _body(user_hbm, item_hbm, uf_hbm, if_hbm, iif_hbm,
          ratings_hbm, logits_hbm,
          uidx_v, iidx_v, u_rows, it_rows, iti_rows, out_r, out_l, sem,
          *, per_w):
    c = lax.axis_index("c")
    s = lax.axis_index("s")
    wid = s * 2 + c
    base = wid * per_w
    nchunk = per_w // CHUNK
    lanes = lax.iota(jnp.int32, LANES)

    def chunk_body(ci, carry):
        off = base + ci * CHUNK
        pltpu.sync_copy(user_hbm.at[pl.ds(off, CHUNK)], uidx_v)
        pltpu.sync_copy(item_hbm.at[pl.ds(off, CHUNK)], iidx_v)
        cp_u = pltpu.async_copy(uf_hbm.at[uidx_v], u_rows, sem)
        cp_t = pltpu.async_copy(if_hbm.at[iidx_v], it_rows, sem)
        cp_i = pltpu.async_copy(iif_hbm.at[iidx_v], iti_rows, sem)
        cp_u.wait()
        cp_t.wait()
        cp_i.wait()

        def group_body(g, gcarry):
            rows = lanes + g * LANES
            acc_r = jnp.zeros((LANES,), jnp.float32)
            acc_l = jnp.zeros((LANES,), jnp.float32)
            for f in range(F):
                fvec = jnp.full((LANES,), f, jnp.int32)
                uf = plsc.load_gather(u_rows, [rows, fvec])
                tf = plsc.load_gather(it_rows, [rows, fvec])
                lf = plsc.load_gather(iti_rows, [rows, fvec])
                acc_r = acc_r + uf * tf
                acc_l = acc_l + uf * lf
            out_r[pl.ds(g * LANES, LANES)] = acc_r
            out_l[pl.ds(g * LANES, LANES)] = acc_l
            return gcarry

        lax.fori_loop(0, CHUNK // LANES, group_body, 0, unroll=False)
        pltpu.sync_copy(out_r, ratings_hbm.at[pl.ds(off, CHUNK)])
        pltpu.sync_copy(out_l, logits_hbm.at[pl.ds(off, CHUNK)])
        return carry

    lax.fori_loop(0, nchunk, chunk_body, 0, unroll=False)


def kernel(user, item, user_factors, item_factors, item_implicit_factors):
    b, l = user.shape
    nb = b * l
    per_w = nb // NW
    assert per_w * NW == nb and per_w % CHUNK == 0

    user_flat = user.reshape(nb)
    item_flat = item.reshape(nb)

    out_sds = jax.ShapeDtypeStruct((nb,), jnp.float32)
    mesh = plsc.VectorSubcoreMesh(core_axis_name="c", subcore_axis_name="s")
    run = pl.kernel(
        functools.partial(_body, per_w=per_w),
        mesh=mesh,
        out_type=(out_sds, out_sds),
        scratch_types=[
            pltpu.VMEM((CHUNK,), jnp.int32),
            pltpu.VMEM((CHUNK,), jnp.int32),
            pltpu.VMEM((CHUNK, F), jnp.float32),
            pltpu.VMEM((CHUNK, F), jnp.float32),
            pltpu.VMEM((CHUNK, F), jnp.float32),
            pltpu.VMEM((CHUNK,), jnp.float32),
            pltpu.VMEM((CHUNK,), jnp.float32),
            pltpu.SemaphoreType.DMA,
        ],
        compiler_params=pltpu.CompilerParams(
            needs_layout_passes=False, use_tc_tiling_on_sc=False),
    )
    ratings, logits = run(user_flat, item_flat, user_factors, item_factors,
                          item_implicit_factors)
    return ratings.reshape(b, l), logits.reshape(b, l)


# speedup vs baseline: 1.8558x; 1.1079x over previous
import functools

import jax
import jax.numpy as jnp
from jax import lax
from jax.experimental import pallas as pl
from jax.experimental.pallas import tpu as pltpu
from jax.experimental.pallas import tpu_sc as plsc

F = 64
NW = 32
CHUNK = 128
LANES = 16


def _body(user_hbm, item_hbm, uf_hbm, if_hbm, iif_hbm,
          ratings_hbm, logits_hbm,
          uidx_v, iidx_v, out_r, out_l,
          u0, t0, i0, u1, t1, i1, s0, s1,
          *, per_w):
    c = lax.axis_index("c")
    s = lax.axis_index("s")
    wid = s * 2 + c
    base = wid * per_w
    nchunk = per_w // CHUNK
    npair = nchunk // 2
    lanes = lax.iota(jnp.int32, LANES)

    pltpu.sync_copy(user_hbm.at[pl.ds(base, per_w)], uidx_v)
    pltpu.sync_copy(item_hbm.at[pl.ds(base, per_w)], iidx_v)

    bufs = ((u0, t0, i0, s0), (u1, t1, i1, s1))

    def fire(ci, slot):
        u_r, t_r, i_r, sem = bufs[slot]
        usl = uidx_v.at[pl.ds(ci * CHUNK, CHUNK)]
        isl = iidx_v.at[pl.ds(ci * CHUNK, CHUNK)]
        pltpu.async_copy(uf_hbm.at[usl], u_r, sem)
        pltpu.async_copy(if_hbm.at[isl], t_r, sem)
        pltpu.async_copy(iif_hbm.at[isl], i_r, sem)

    def wait_slot(slot):
        u_r, t_r, i_r, sem = bufs[slot]
        pltpu.make_async_copy(uf_hbm.at[pl.ds(0, CHUNK)], u_r, sem).wait()
        pltpu.make_async_copy(if_hbm.at[pl.ds(0, CHUNK)], t_r, sem).wait()
        pltpu.make_async_copy(iif_hbm.at[pl.ds(0, CHUNK)], i_r, sem).wait()

    def compute(ci, slot):
        u_r, t_r, i_r, _ = bufs[slot]

        def group_body(g, gcarry):
            rows = lanes + g * LANES
            acc_r = jnp.zeros((LANES,), jnp.float32)
            acc_l = jnp.zeros((LANES,), jnp.float32)
            for f in range(F):
                fvec = jnp.full((LANES,), f, jnp.int32)
                uf = plsc.load_gather(u_r, [rows, fvec])
                tf = plsc.load_gather(t_r, [rows, fvec])
                lf = plsc.load_gather(i_r, [rows, fvec])
                acc_r = acc_r + uf * tf
                acc_l = acc_l + uf * lf
            o = ci * CHUNK + g * LANES
            out_r[pl.ds(o, LANES)] = acc_r
            out_l[pl.ds(o, LANES)] = acc_l
            return gcarry

        lax.fori_loop(0, CHUNK // LANES, group_body, 0, unroll=False)

    fire(0, 0)

    def pair_body(p, carry):
        ci0 = 2 * p
        fire(ci0 + 1, 1)
        wait_slot(0)
        compute(ci0, 0)

        @pl.when(p + 1 < npair)
        def _():
            fire(ci0 + 2, 0)

        wait_slot(1)
        compute(ci0 + 1, 1)
        return carry

    lax.fori_loop(0, npair, pair_body, 0, unroll=False)

    pltpu.sync_copy(out_r, ratings_hbm.at[pl.ds(base, per_w)])
    pltpu.sync_copy(out_l, logits_hbm.at[pl.ds(base, per_w)])


def kernel(user, item, user_factors, item_factors, item_implicit_factors):
    b, l = user.shape
    nb = b * l
    per_w = nb // NW
    assert per_w * NW == nb and per_w % (2 * CHUNK) == 0

    user_flat = user.reshape(nb)
    item_flat = item.reshape(nb)

    out_sds = jax.ShapeDtypeStruct((nb,), jnp.float32)
    mesh = plsc.VectorSubcoreMesh(core_axis_name="c", subcore_axis_name="s")
    row_buf = pltpu.VMEM((CHUNK, F), jnp.float32)
    run = pl.kernel(
        functools.partial(_body, per_w=per_w),
        mesh=mesh,
        out_type=(out_sds, out_sds),
        scratch_types=[
            pltpu.VMEM((per_w,), jnp.int32),
            pltpu.VMEM((per_w,), jnp.int32),
            pltpu.VMEM((per_w,), jnp.float32),
            pltpu.VMEM((per_w,), jnp.float32),
            row_buf, row_buf, row_buf,
            row_buf, row_buf, row_buf,
            pltpu.SemaphoreType.DMA,
            pltpu.SemaphoreType.DMA,
        ],
        compiler_params=pltpu.CompilerParams(
            needs_layout_passes=False, use_tc_tiling_on_sc=False),
    )
    ratings, logits = run(user_flat, item_flat, user_factors, item_factors,
                          item_implicit_factors)
    return ratings.reshape(b, l), logits.reshape(b, l)
